# baseline (speedup 1.0000x reference)
import jax
import jax.numpy as jnp
from jax.experimental import pallas as pl
from jax.experimental.pallas import tpu as pltpu

B, T, D = 64, 576, 768
E, R = 8, 192
RB = 8
RSTEPS = B // RB


def _routing_kernel(tokens_ref, gate_W_ref, gate_b_ref,
                    logits_ref, sel_ref, top1_ref, ew_ref, imp_ref, load_ref,
                    pooled_ref):
    b = pl.program_id(0)
    pooled_ref[pl.ds(b * RB, RB), :] = jnp.mean(tokens_ref[...], axis=1)

    @pl.when(b == RSTEPS - 1)
    def _finish():
        pooled = pooled_ref[...]
        logits = jnp.dot(pooled, gate_W_ref[...],
                         preferred_element_type=jnp.float32) + gate_b_ref[...]
        m = jnp.max(logits, axis=-1, keepdims=True)
        p = jnp.exp(logits - m)
        p = p / jnp.sum(p, axis=-1, keepdims=True)
        top1 = jnp.max(p, axis=-1, keepdims=True)
        iota_e = jax.lax.broadcasted_iota(jnp.int32, (B, E), 1)
        sel = jnp.min(jnp.where(p == top1, iota_e, E), axis=-1, keepdims=True)
        onehot = (iota_e == sel).astype(jnp.float32)
        logits_ref[...] = logits
        sel_ref[...] = sel
        top1_ref[...] = top1
        ew_ref[...] = onehot * top1
        imp_ref[...] = jnp.sum(onehot * top1, axis=0, keepdims=True)
        load_ref[...] = jnp.sum(onehot, axis=0, keepdims=True) / B


def _adapter_kernel(sel_sp, t1_sp, tokens_ref, wd_ref, wu_ref, bd_ref, bu_ref,
                    out_ref):
    b = pl.program_id(0)
    e = sel_sp[b]
    x = tokens_ref[0]
    h = jnp.dot(x.astype(jnp.bfloat16), wd_ref[e],
                preferred_element_type=jnp.float32) + bd_ref[e, :][None, :]
    h = jax.nn.gelu(h)
    y = jnp.dot(h.astype(jnp.bfloat16), wu_ref[e],
                preferred_element_type=jnp.float32) + bu_ref[e, :][None, :]
    out_ref[0] = (x + y) * t1_sp[b]


@jax.jit
def kernel(tokens, spatial_shape, gate_W, gate_b, W_down, b_down, W_up, b_up):
    del spatial_shape
    logits, sel, top1, ew, imp, load = pl.pallas_call(
        _routing_kernel,
        grid=(RSTEPS,),
        in_specs=[
            pl.BlockSpec((RB, T, D), lambda b: (b, 0, 0)),
            pl.BlockSpec((D, E), lambda b: (0, 0)),
            pl.BlockSpec((1, E), lambda b: (0, 0)),
        ],
        out_specs=[
            pl.BlockSpec((B, E), lambda b: (0, 0)),
            pl.BlockSpec((B, 1), lambda b: (0, 0)),
            pl.BlockSpec((B, 1), lambda b: (0, 0)),
            pl.BlockSpec((B, E), lambda b: (0, 0)),
            pl.BlockSpec((1, E), lambda b: (0, 0)),
            pl.BlockSpec((1, E), lambda b: (0, 0)),
        ],
        out_shape=[
            jax.ShapeDtypeStruct((B, E), jnp.float32),
            jax.ShapeDtypeStruct((B, 1), jnp.int32),
            jax.ShapeDtypeStruct((B, 1), jnp.float32),
            jax.ShapeDtypeStruct((B, E), jnp.float32),
            jax.ShapeDtypeStruct((1, E), jnp.float32),
            jax.ShapeDtypeStruct((1, E), jnp.float32),
        ],
        scratch_shapes=[pltpu.VMEM((B, D), jnp.float32)],
        compiler_params=pltpu.CompilerParams(
            dimension_semantics=("arbitrary",)),
    )(tokens, gate_W, gate_b.reshape(1, E))

    sel_flat = sel.reshape(B)
    t1_flat = top1.reshape(B)

    grid_spec = pltpu.PrefetchScalarGridSpec(
        num_scalar_prefetch=2,
        grid=(B,),
        in_specs=[
            pl.BlockSpec((1, T, D), lambda b, s, t: (b, 0, 0)),
            pl.BlockSpec((E, D, R), lambda b, s, t: (0, 0, 0)),
            pl.BlockSpec((E, R, D), lambda b, s, t: (0, 0, 0)),
            pl.BlockSpec((E, R), lambda b, s, t: (0, 0)),
            pl.BlockSpec((E, D), lambda b, s, t: (0, 0)),
        ],
        out_specs=pl.BlockSpec((1, T, D), lambda b, s, t: (b, 0, 0)),
    )
    weighted = pl.pallas_call(
        _adapter_kernel,
        grid_spec=grid_spec,
        out_shape=jax.ShapeDtypeStruct((B, T, D), jnp.float32),
        compiler_params=pltpu.CompilerParams(
            dimension_semantics=("arbitrary",)),
    )(sel_flat, t1_flat, tokens, W_down.astype(jnp.bfloat16),
      W_up.astype(jnp.bfloat16), b_down, b_up)

    return (weighted, logits, sel, ew, imp.reshape(E), load.reshape(E))


# speedup vs baseline: 1.2523x; 1.2523x over previous
import jax
import jax.numpy as jnp
from jax.experimental import pallas as pl
from jax.experimental.pallas import tpu as pltpu

B, T, D = 64, 576, 768
E, R = 8, 192
RB = 8
RSTEPS = B // RB
SB = 4


def _routing_kernel(tokens_ref, gate_W_ref, gate_b_ref,
                    logits_ref, sel_ref, top1_ref, ew_ref, imp_ref, load_ref,
                    pooled_ref):
    b = pl.program_id(0)
    pooled_ref[pl.ds(b * RB, RB), :] = jnp.mean(tokens_ref[...], axis=1)

    @pl.when(b == RSTEPS - 1)
    def _finish():
        pooled = pooled_ref[...]
        logits = jnp.dot(pooled, gate_W_ref[...],
                         preferred_element_type=jnp.float32) + gate_b_ref[...]
        m = jnp.max(logits, axis=-1, keepdims=True)
        p = jnp.exp(logits - m)
        p = p / jnp.sum(p, axis=-1, keepdims=True)
        top1 = jnp.max(p, axis=-1, keepdims=True)
        iota_e = jax.lax.broadcasted_iota(jnp.int32, (B, E), 1)
        sel = jnp.min(jnp.where(p == top1, iota_e, E), axis=-1, keepdims=True)
        onehot = (iota_e == sel).astype(jnp.float32)
        logits_ref[...] = logits
        sel_ref[...] = sel
        top1_ref[...] = top1
        ew_ref[...] = onehot * top1
        imp_ref[...] = jnp.sum(onehot * top1, axis=0, keepdims=True)
        load_ref[...] = jnp.sum(onehot, axis=0, keepdims=True) / B


def _adapter_kernel(sel_sp, t1_sp, tokens_ref, wd_ref, wu_ref, bd_ref, bu_ref,
                    out_ref):
    g = pl.program_id(0)
    for j in range(SB):
        e = sel_sp[g * SB + j]
        x = tokens_ref[j]
        h = jnp.dot(x.astype(jnp.bfloat16), wd_ref[e],
                    preferred_element_type=jnp.float32) + bd_ref[e, :][None, :]
        h = jax.nn.gelu(h)
        y = jnp.dot(h.astype(jnp.bfloat16), wu_ref[e],
                    preferred_element_type=jnp.float32) + bu_ref[e, :][None, :]
        out_ref[j] = (x + y) * t1_sp[g * SB + j]


@jax.jit
def kernel(tokens, spatial_shape, gate_W, gate_b, W_down, b_down, W_up, b_up):
    del spatial_shape
    logits, sel, top1, ew, imp, load = pl.pallas_call(
        _routing_kernel,
        grid=(RSTEPS,),
        in_specs=[
            pl.BlockSpec((RB, T, D), lambda b: (b, 0, 0)),
            pl.BlockSpec((D, E), lambda b: (0, 0)),
            pl.BlockSpec((1, E), lambda b: (0, 0)),
        ],
        out_specs=[
            pl.BlockSpec((B, E), lambda b: (0, 0)),
            pl.BlockSpec((B, 1), lambda b: (0, 0)),
            pl.BlockSpec((B, 1), lambda b: (0, 0)),
            pl.BlockSpec((B, E), lambda b: (0, 0)),
            pl.BlockSpec((1, E), lambda b: (0, 0)),
            pl.BlockSpec((1, E), lambda b: (0, 0)),
        ],
        out_shape=[
            jax.ShapeDtypeStruct((B, E), jnp.float32),
            jax.ShapeDtypeStruct((B, 1), jnp.int32),
            jax.ShapeDtypeStruct((B, 1), jnp.float32),
            jax.ShapeDtypeStruct((B, E), jnp.float32),
            jax.ShapeDtypeStruct((1, E), jnp.float32),
            jax.ShapeDtypeStruct((1, E), jnp.float32),
        ],
        scratch_shapes=[pltpu.VMEM((B, D), jnp.float32)],
        compiler_params=pltpu.CompilerParams(
            dimension_semantics=("arbitrary",)),
    )(tokens, gate_W, gate_b.reshape(1, E))

    sel_flat = sel.reshape(B)
    t1_flat = top1.reshape(B)

    grid_spec = pltpu.PrefetchScalarGridSpec(
        num_scalar_prefetch=2,
        grid=(B // SB,),
        in_specs=[
            pl.BlockSpec((SB, T, D), lambda b, s, t: (b, 0, 0)),
            pl.BlockSpec((E, D, R), lambda b, s, t: (0, 0, 0)),
            pl.BlockSpec((E, R, D), lambda b, s, t: (0, 0, 0)),
            pl.BlockSpec((E, R), lambda b, s, t: (0, 0)),
            pl.BlockSpec((E, D), lambda b, s, t: (0, 0)),
        ],
        out_specs=pl.BlockSpec((SB, T, D), lambda b, s, t: (b, 0, 0)),
    )
    weighted = pl.pallas_call(
        _adapter_kernel,
        grid_spec=grid_spec,
        out_shape=jax.ShapeDtypeStruct((B, T, D), jnp.float32),
        compiler_params=pltpu.CompilerParams(
            dimension_semantics=("arbitrary",)),
    )(sel_flat, t1_flat, tokens, W_down.astype(jnp.bfloat16),
      W_up.astype(jnp.bfloat16), b_down, b_up)

    return (weighted, logits, sel, ew, imp.reshape(E), load.reshape(E))


# speedup vs baseline: 1.2920x; 1.0317x over previous
import jax
import jax.numpy as jnp
from jax.experimental import pallas as pl
from jax.experimental.pallas import tpu as pltpu

B, T, D = 64, 576, 768
E, R = 8, 192
RB = 4
RSTEPS = B // RB
SB = 4
ASTEPS = B // SB


def _moe_kernel(tokens_r_ref, tokens_a_ref, gate_W_ref, gate_b_ref,
                wd_ref, wu_ref, bd_ref, bu_ref,
                out_ref, logits_ref, sel_ref, top1_ref, ew_ref, imp_ref,
                load_ref,
                pooled_ref, selt_ref, t1t_ref, sel_smem, t1_smem, sem):
    i = pl.program_id(0)

    @pl.when(i < RSTEPS)
    def _route():
        pooled_ref[i] = jnp.mean(tokens_r_ref[...], axis=1)

    @pl.when(i == RSTEPS - 1)
    def _finish_route():
        pooled = pooled_ref[...].reshape(B, D)
        logits = jnp.dot(pooled, gate_W_ref[...],
                         preferred_element_type=jnp.float32) + gate_b_ref[...]
        m = jnp.max(logits, axis=-1, keepdims=True)
        p = jnp.exp(logits - m)
        p = p / jnp.sum(p, axis=-1, keepdims=True)
        top1 = jnp.max(p, axis=-1, keepdims=True)
        iota_e = jax.lax.broadcasted_iota(jnp.int32, (B, E), 1)
        sel = jnp.min(jnp.where(p == top1, iota_e, E), axis=-1, keepdims=True)
        onehot = (iota_e == sel).astype(jnp.float32)
        logits_ref[...] = logits
        sel_ref[...] = sel
        top1_ref[...] = top1
        ew_ref[...] = onehot * top1
        imp_ref[...] = jnp.sum(onehot * top1, axis=0, keepdims=True)
        load_ref[...] = jnp.sum(onehot, axis=0, keepdims=True) / B
        selt_ref[...] = sel.reshape(1, B)
        t1t_ref[...] = top1.reshape(1, B)
        cp1 = pltpu.make_async_copy(selt_ref, sel_smem, sem)
        cp1.start()
        cp1.wait()
        cp2 = pltpu.make_async_copy(t1t_ref, t1_smem, sem)
        cp2.start()
        cp2.wait()

    @pl.when(i >= RSTEPS)
    def _adapt():
        g = i - RSTEPS
        for j in range(SB):
            e = sel_smem[0, g * SB + j]
            t1 = t1_smem[0, g * SB + j]
            x = tokens_a_ref[j]
            h = jnp.dot(x.astype(jnp.bfloat16), wd_ref[e],
                        preferred_element_type=jnp.float32) + bd_ref[e, :][None, :]
            h = jax.nn.gelu(h)
            y = jnp.dot(h.astype(jnp.bfloat16), wu_ref[e],
                        preferred_element_type=jnp.float32) + bu_ref[e, :][None, :]
            out_ref[j] = (x + y) * t1


@jax.jit
def kernel(tokens, spatial_shape, gate_W, gate_b, W_down, b_down, W_up, b_up):
    del spatial_shape
    out, logits, sel, top1, ew, imp, load = pl.pallas_call(
        _moe_kernel,
        grid=(RSTEPS + ASTEPS,),
        in_specs=[
            pl.BlockSpec((RB, T, D),
                         lambda i: (jnp.minimum(i, RSTEPS - 1), 0, 0)),
            pl.BlockSpec((SB, T, D),
                         lambda i: (jnp.maximum(i - RSTEPS, 0), 0, 0)),
            pl.BlockSpec((D, E), lambda i: (0, 0)),
            pl.BlockSpec((1, E), lambda i: (0, 0)),
            pl.BlockSpec((E, D, R), lambda i: (0, 0, 0)),
            pl.BlockSpec((E, R, D), lambda i: (0, 0, 0)),
            pl.BlockSpec((E, R), lambda i: (0, 0)),
            pl.BlockSpec((E, D), lambda i: (0, 0)),
        ],
        out_specs=[
            pl.BlockSpec((SB, T, D),
                         lambda i: (jnp.maximum(i - RSTEPS, 0), 0, 0)),
            pl.BlockSpec((B, E), lambda i: (0, 0)),
            pl.BlockSpec((B, 1), lambda i: (0, 0)),
            pl.BlockSpec((B, 1), lambda i: (0, 0)),
            pl.BlockSpec((B, E), lambda i: (0, 0)),
            pl.BlockSpec((1, E), lambda i: (0, 0)),
            pl.BlockSpec((1, E), lambda i: (0, 0)),
        ],
        out_shape=[
            jax.ShapeDtypeStruct((B, T, D), jnp.float32),
            jax.ShapeDtypeStruct((B, E), jnp.float32),
            jax.ShapeDtypeStruct((B, 1), jnp.int32),
            jax.ShapeDtypeStruct((B, 1), jnp.float32),
            jax.ShapeDtypeStruct((B, E), jnp.float32),
            jax.ShapeDtypeStruct((1, E), jnp.float32),
            jax.ShapeDtypeStruct((1, E), jnp.float32),
        ],
        scratch_shapes=[
            pltpu.VMEM((RSTEPS, RB, D), jnp.float32),
            pltpu.VMEM((1, B), jnp.int32),
            pltpu.VMEM((1, B), jnp.float32),
            pltpu.SMEM((1, B), jnp.int32),
            pltpu.SMEM((1, B), jnp.float32),
            pltpu.SemaphoreType.DMA,
        ],
        compiler_params=pltpu.CompilerParams(
            dimension_semantics=("arbitrary",)),
    )(tokens, tokens, gate_W, gate_b.reshape(1, E),
      W_down.astype(jnp.bfloat16), W_up.astype(jnp.bfloat16), b_down, b_up)

    return (out, logits, sel, ew, imp.reshape(E), load.reshape(E))
